# baseline (speedup 1.0000x reference)
import functools

import jax
import jax.numpy as jnp
from jax import lax
from jax.experimental import pallas as pl
from jax.experimental.pallas import tpu as pltpu
from jax.experimental.pallas import tpu_sc as plsc

_NC = 2
_NS = 16
_NW = _NC * _NS

_CH = 512
_G = 128
_PAD = 72
_W = 72

_CHT = 4096


@functools.lru_cache(maxsize=None)
def _build_gather(n_rows: int):
    assert n_rows % (_NW * _CH) == 0
    n_per_w = n_rows // _NW
    n_chunks = n_per_w // _CH
    mesh = plsc.VectorSubcoreMesh(core_axis_name="c", subcore_axis_name="s")

    @functools.partial(
        pl.kernel,
        mesh=mesh,
        compiler_params=pltpu.CompilerParams(use_tc_tiling_on_sc=False),
        out_type=jax.ShapeDtypeStruct((n_rows, _PAD), jnp.float32),
        scratch_types=[
            pltpu.VMEM((_CH,), jnp.int32),
            pltpu.VMEM((_CH, _PAD), jnp.float32),
            pltpu.SemaphoreType.DMA,
        ],
    )
    def gather_kernel(idx_hbm, table_hbm, out_hbm, idx_v, rows_v, sem):
        wid = lax.axis_index("s") * _NC + lax.axis_index("c")
        base = wid * n_per_w

        def chunk_body(c, carry):
            off = base + c * _CH
            pltpu.sync_copy(idx_hbm.at[pl.ds(off, _CH)], idx_v)
            copies = []
            for g in range(_CH // _G):
                copies.append(
                    pltpu.async_copy(
                        table_hbm.at[idx_v.at[pl.ds(g * _G, _G)]],
                        rows_v.at[pl.ds(g * _G, _G)],
                        sem,
                    )
                )
            for cp in copies:
                cp.wait()
            pltpu.sync_copy(
                rows_v.at[pl.ds(0, _CH), pl.ds(0, _W)],
                out_hbm.at[pl.ds(off, _CH), pl.ds(0, _W)],
            )
            return carry

        lax.fori_loop(0, n_chunks, chunk_body, 0)

    return gather_kernel


_TB = 1024


@functools.lru_cache(maxsize=None)
def _build_transpose_pad(n_nodes: int, n_cols: int):
    def transpose_body(in_ref, out_ref):
        out_ref[:, :n_cols] = in_ref[...].T

    return pl.pallas_call(
        transpose_body,
        grid=(pl.cdiv(n_nodes, _TB),),
        in_specs=[pl.BlockSpec((n_cols, _TB), lambda i: (0, i))],
        out_specs=pl.BlockSpec((_TB, _PAD), lambda i: (i, 0)),
        out_shape=jax.ShapeDtypeStruct((n_nodes, _PAD), jnp.float32),
    )


@functools.lru_cache(maxsize=None)
def _build_repack(n_rows: int, n_cols: int):
    assert n_rows % _CHT == 0

    def repack_body(in_ref, out_ref):
        out_ref[...] = in_ref[:, :n_cols]

    return pl.pallas_call(
        repack_body,
        grid=(n_rows // _CHT,),
        in_specs=[pl.BlockSpec((_CHT, _PAD), lambda i: (i, 0))],
        out_specs=pl.BlockSpec((_CHT, n_cols), lambda i: (i, 0)),
        out_shape=jax.ShapeDtypeStruct((n_rows, n_cols), jnp.float32),
    )


def kernel(x, embedding):
    b, h = x.shape
    n = b * h
    n_nodes, n_cols = embedding.shape
    idx = x.reshape(n).astype(jnp.int32)
    table_pad = _build_transpose_pad(n_nodes, n_cols)(embedding.T)
    out_pad = _build_gather(n)(idx, table_pad)
    out = _build_repack(n, n_cols)(out_pad)
    return out.reshape(b, h, n_cols)


# speedup vs baseline: 1.6137x; 1.6137x over previous
import functools

import jax
import jax.numpy as jnp
from jax import lax
from jax.experimental import pallas as pl
from jax.experimental.pallas import tpu as pltpu
from jax.experimental.pallas import tpu_sc as plsc

_NC = 2
_NS = 16
_NW = _NC * _NS

_CH = 512
_G = 128
_PAD = 128
_W = 72

_CHT = 4096


@functools.lru_cache(maxsize=None)
def _build_gather(n_rows: int):
    assert n_rows % (_NW * _CH) == 0
    n_per_w = n_rows // _NW
    n_chunks = n_per_w // _CH
    mesh = plsc.VectorSubcoreMesh(core_axis_name="c", subcore_axis_name="s")

    @functools.partial(
        pl.kernel,
        mesh=mesh,
        compiler_params=pltpu.CompilerParams(use_tc_tiling_on_sc=False),
        out_type=jax.ShapeDtypeStruct((n_rows, _PAD), jnp.float32),
        scratch_types=[
            pltpu.VMEM((_CH,), jnp.int32),
            pltpu.VMEM((_CH, _PAD), jnp.float32),
            pltpu.SemaphoreType.DMA,
        ],
    )
    def gather_kernel(idx_hbm, table_hbm, out_hbm, idx_v, rows_v, sem):
        wid = lax.axis_index("s") * _NC + lax.axis_index("c")
        base = wid * n_per_w

        def chunk_body(c, carry):
            off = base + c * _CH
            pltpu.sync_copy(idx_hbm.at[pl.ds(off, _CH)], idx_v)
            copies = []
            for g in range(_CH // _G):
                copies.append(
                    pltpu.async_copy(
                        table_hbm.at[idx_v.at[pl.ds(g * _G, _G)]],
                        rows_v.at[pl.ds(g * _G, _G)],
                        sem,
                    )
                )
            for cp in copies:
                cp.wait()
            pltpu.sync_copy(
                rows_v.at[pl.ds(0, _CH), pl.ds(0, _W)],
                out_hbm.at[pl.ds(off, _CH), pl.ds(0, _W)],
            )
            return carry

        lax.fori_loop(0, n_chunks, chunk_body, 0)

    return gather_kernel


_TB = 1024


@functools.lru_cache(maxsize=None)
def _build_transpose_pad(n_nodes: int, n_cols: int):
    def transpose_body(in_ref, out_ref):
        out_ref[:, :n_cols] = in_ref[...].T

    return pl.pallas_call(
        transpose_body,
        grid=(pl.cdiv(n_nodes, _TB),),
        in_specs=[pl.BlockSpec((n_cols, _TB), lambda i: (0, i))],
        out_specs=pl.BlockSpec((_TB, _PAD), lambda i: (i, 0)),
        out_shape=jax.ShapeDtypeStruct((n_nodes, _PAD), jnp.float32),
    )


_RB = 256
_RH = 8


@functools.lru_cache(maxsize=None)
def _build_repack_t(n_batch: int, n_hist: int, n_cols: int):
    assert n_batch % _RB == 0 and n_hist % _RH == 0

    def repack_body(in_ref, out_ref):
        for h in range(_RH):
            out_ref[:, h, :] = in_ref[:, h, :].T[:n_cols]

    return pl.pallas_call(
        repack_body,
        grid=(n_batch // _RB, n_hist // _RH),
        in_specs=[pl.BlockSpec((_RB, _RH, _PAD), lambda i, j: (i, j, 0))],
        out_specs=pl.BlockSpec((n_cols, _RH, _RB), lambda i, j: (0, j, i)),
        out_shape=jax.ShapeDtypeStruct((n_cols, n_hist, n_batch), jnp.float32),
    )


def kernel(x, embedding):
    b, h = x.shape
    n = b * h
    n_nodes, n_cols = embedding.shape
    idx = x.reshape(n).astype(jnp.int32)
    table_pad = _build_transpose_pad(n_nodes, n_cols)(embedding.T)
    out_pad = _build_gather(n)(idx, table_pad)
    out_t = _build_repack_t(b, h, n_cols)(out_pad.reshape(b, h, _PAD))
    return out_t.transpose(2, 1, 0)
